# initial kernel scaffold (unmeasured)
import jax
import jax.numpy as jnp
from jax import lax
from jax.experimental import pallas as pl
from jax.experimental.pallas import tpu as pltpu


def kernel(
    x,
):
    def body(*refs):
        pass

    out_shape = jax.ShapeDtypeStruct(..., jnp.float32)
    return pl.pallas_call(body, out_shape=out_shape)(...)


# baseline (device time: 16340 ns/iter reference)
import jax
import jax.numpy as jnp
from jax import lax
from jax.experimental import pallas as pl
from jax.experimental.pallas import tpu as pltpu

N_DEV = 32


def _local_cumprod(c, m, n):
    d = 1
    while d < m:
        shifted = jnp.concatenate(
            [jnp.ones((d, n), c.dtype), c[: m - d, :]], axis=0
        )
        c = c * shifted
        d *= 2
    return c


def kernel(x):
    m, n = x.shape

    def body(x_ref, out_ref, comm_ref, send_buf, send_sem, recv_sem):
        my_i = lax.axis_index("i")

        c = _local_cumprod(x_ref[...], m, n)
        total = c[m - 1 : m, :]

        @pl.when(my_i > 0)
        def _():
            recv = pltpu.make_async_remote_copy(
                src_ref=send_buf,
                dst_ref=comm_ref,
                send_sem=send_sem,
                recv_sem=recv_sem,
                device_id=(my_i - 1,),
                device_id_type=pl.DeviceIdType.MESH,
            )
            recv.wait_recv()

        @pl.when(my_i == 0)
        def _():
            comm_ref[...] = jnp.ones((1, n), dtype=x_ref.dtype)

        prefix = comm_ref[...]

        @pl.when(my_i < N_DEV - 1)
        def _():
            send_buf[...] = prefix * total
            send = pltpu.make_async_remote_copy(
                src_ref=send_buf,
                dst_ref=comm_ref,
                send_sem=send_sem,
                recv_sem=recv_sem,
                device_id=(my_i + 1,),
                device_id_type=pl.DeviceIdType.MESH,
            )
            send.start()
            send.wait_send()

        out_ref[...] = c * prefix

    return pl.pallas_call(
        body,
        out_shape=jax.ShapeDtypeStruct((m, n), x.dtype),
        in_specs=[pl.BlockSpec(memory_space=pltpu.VMEM)],
        out_specs=pl.BlockSpec(memory_space=pltpu.VMEM),
        scratch_shapes=[
            pltpu.VMEM((1, n), jnp.float32),
            pltpu.VMEM((1, n), jnp.float32),
            pltpu.SemaphoreType.DMA,
            pltpu.SemaphoreType.DMA,
        ],
    )(x)


# device time: 13733 ns/iter; 1.1898x vs baseline; 1.1898x over previous
import jax
import jax.numpy as jnp
from jax import lax
from jax.experimental import pallas as pl
from jax.experimental.pallas import tpu as pltpu

N_DEV = 32


def _local_cumprod(c, m, n):
    d = 1
    while d < m:
        shifted = jnp.concatenate(
            [jnp.ones((d, n), c.dtype), c[: m - d, :]], axis=0
        )
        c = c * shifted
        d *= 2
    return c


def kernel(x):
    m, n = x.shape

    def body(x_ref, out_ref, comm_ref, send_buf, send_sems, recv_sems):
        my_i = lax.axis_index("i")

        c = _local_cumprod(x_ref[...], m, n)
        send_buf[...] = c[m - 1 : m, :]

        for j in range(N_DEV):
            @pl.when(my_i < j)
            def _(j=j):
                send = pltpu.make_async_remote_copy(
                    src_ref=send_buf,
                    dst_ref=comm_ref.at[my_i],
                    send_sem=send_sems.at[j],
                    recv_sem=recv_sems.at[my_i],
                    device_id=(j,),
                    device_id_type=pl.DeviceIdType.MESH,
                )
                send.start()

        for idx in range(N_DEV):
            @pl.when(idx < my_i)
            def _(idx=idx):
                recv = pltpu.make_async_remote_copy(
                    src_ref=send_buf,
                    dst_ref=comm_ref.at[idx],
                    send_sem=send_sems.at[idx],
                    recv_sem=recv_sems.at[idx],
                    device_id=(idx,),
                    device_id_type=pl.DeviceIdType.MESH,
                )
                recv.wait_recv()

        gathered = comm_ref[...]
        row = lax.broadcasted_iota(jnp.int32, (N_DEV, 1, n), 0)
        masked = jnp.where(row < my_i, gathered, jnp.ones_like(gathered))
        half = N_DEV
        while half > 1:
            half //= 2
            masked = masked[:half] * masked[half : 2 * half]
        prefix = masked[0]

        out_ref[...] = c * prefix

        for j in range(N_DEV):
            @pl.when(my_i < j)
            def _(j=j):
                send = pltpu.make_async_remote_copy(
                    src_ref=send_buf,
                    dst_ref=comm_ref.at[my_i],
                    send_sem=send_sems.at[j],
                    recv_sem=recv_sems.at[my_i],
                    device_id=(j,),
                    device_id_type=pl.DeviceIdType.MESH,
                )
                send.wait_send()

    return pl.pallas_call(
        body,
        out_shape=jax.ShapeDtypeStruct((m, n), x.dtype),
        in_specs=[pl.BlockSpec(memory_space=pltpu.VMEM)],
        out_specs=pl.BlockSpec(memory_space=pltpu.VMEM),
        scratch_shapes=[
            pltpu.VMEM((N_DEV, 1, n), jnp.float32),
            pltpu.VMEM((1, n), jnp.float32),
            pltpu.SemaphoreType.DMA((N_DEV,)),
            pltpu.SemaphoreType.DMA((N_DEV,)),
        ],
    )(x)


# device time: 7078 ns/iter; 2.3086x vs baseline; 1.9402x over previous
import jax
import jax.numpy as jnp
from jax import lax
from jax.experimental import pallas as pl
from jax.experimental.pallas import tpu as pltpu

N_DEV = 32
N_ROUNDS = 5


def _tree_prod(c, m, n):
    rows = m
    while rows > 1:
        rows //= 2
        c = c[:rows] * c[rows : 2 * rows]
    return c


def _local_cumprod(c, m, n):
    d = 1
    while d < m:
        shifted = jnp.concatenate(
            [jnp.ones((d, n), c.dtype), c[: m - d, :]], axis=0
        )
        c = c * shifted
        d *= 2
    return c


def kernel(x):
    m, n = x.shape

    def body(
        x_ref, out_ref, comm_ref, send_bufs, send_sems, recv_sems, ready_sems
    ):
        my_i = lax.axis_index("i")

        for r in range(N_ROUNDS):
            d = 1 << r
            @pl.when(my_i >= d)
            def _(r=r, d=d):
                pl.semaphore_signal(
                    ready_sems.at[r],
                    inc=1,
                    device_id=(my_i - d,),
                    device_id_type=pl.DeviceIdType.MESH,
                )

        barrier_sem = pltpu.get_barrier_semaphore()
        @pl.when(my_i > 0)
        def _():
            pl.semaphore_signal(
                barrier_sem, inc=1,
                device_id=(my_i - 1,), device_id_type=pl.DeviceIdType.MESH,
            )
        @pl.when(my_i < N_DEV - 1)
        def _():
            pl.semaphore_signal(
                barrier_sem, inc=1,
                device_id=(my_i + 1,), device_id_type=pl.DeviceIdType.MESH,
            )

        s = _tree_prod(x_ref[...], m, n)
        p = jnp.ones((1, n), dtype=x_ref.dtype)

        @pl.when(my_i > 0)
        def _():
            pl.semaphore_wait(barrier_sem, 1)
        @pl.when(my_i < N_DEV - 1)
        def _():
            pl.semaphore_wait(barrier_sem, 1)

        c = None
        for r in range(N_ROUNDS):
            d = 1 << r

            @pl.when(my_i + d < N_DEV)
            def _(r=r, d=d, s=s):
                pl.semaphore_wait(ready_sems.at[r], 1)
                send_bufs[r] = s
                send = pltpu.make_async_remote_copy(
                    src_ref=send_bufs.at[r],
                    dst_ref=comm_ref.at[r],
                    send_sem=send_sems.at[r],
                    recv_sem=recv_sems.at[r],
                    device_id=(my_i + d,),
                    device_id_type=pl.DeviceIdType.MESH,
                )
                send.start()

            if r == 0:
                c = _local_cumprod(x_ref[...], m, n)

            @pl.when(my_i >= d)
            def _(r=r, d=d):
                recv = pltpu.make_async_remote_copy(
                    src_ref=send_bufs.at[r],
                    dst_ref=comm_ref.at[r],
                    send_sem=send_sems.at[r],
                    recv_sem=recv_sems.at[r],
                    device_id=(my_i - d,),
                    device_id_type=pl.DeviceIdType.MESH,
                )
                recv.wait_recv()

            v = comm_ref[r]
            got = my_i >= d
            p = jnp.where(got, p * v, p)
            s = jnp.where(got, s * v, s)

        out_ref[...] = c * p

        for r in range(N_ROUNDS):
            d = 1 << r
            @pl.when(my_i + d < N_DEV)
            def _(r=r, d=d):
                send = pltpu.make_async_remote_copy(
                    src_ref=send_bufs.at[r],
                    dst_ref=comm_ref.at[r],
                    send_sem=send_sems.at[r],
                    recv_sem=recv_sems.at[r],
                    device_id=(my_i + d,),
                    device_id_type=pl.DeviceIdType.MESH,
                )
                send.wait_send()

    return pl.pallas_call(
        body,
        out_shape=jax.ShapeDtypeStruct((m, n), x.dtype),
        in_specs=[pl.BlockSpec(memory_space=pltpu.VMEM)],
        out_specs=pl.BlockSpec(memory_space=pltpu.VMEM),
        scratch_shapes=[
            pltpu.VMEM((N_ROUNDS, 1, n), jnp.float32),
            pltpu.VMEM((N_ROUNDS, 1, n), jnp.float32),
            pltpu.SemaphoreType.DMA((N_ROUNDS,)),
            pltpu.SemaphoreType.DMA((N_ROUNDS,)),
            pltpu.SemaphoreType.REGULAR((N_ROUNDS,)),
        ],
        compiler_params=pltpu.CompilerParams(collective_id=0),
    )(x)
